# Initial kernel scaffold; baseline (speedup 1.0000x reference)
import jax
import jax.numpy as jnp
from jax.experimental import pallas as pl


def kernel(X_pooled, X_original, A, idx):
    raise NotImplementedError("write your pallas kernel here")


# speedup vs baseline: 148.3264x; 148.3264x over previous
import functools

import jax
import jax.numpy as jnp
from jax import lax
from jax.experimental import pallas as pl
from jax.experimental.pallas import tpu as pltpu
from jax.experimental.pallas import tpu_sc as plsc

L = 16


def _build_sc_kernel(R, K, V, NB, B):
    info = plsc.get_sparse_core_info()
    NC, NS = info.num_cores, info.num_subcores
    NW = NC * NS
    rows_per_w = R // NW
    n_groups = rows_per_w // NB
    n_iters = n_groups // 2
    kc = K // L

    mesh = plsc.VectorSubcoreMesh(core_axis_name="c", subcore_axis_name="s")

    @functools.partial(
        pl.kernel,
        mesh=mesh,
        out_type=jax.ShapeDtypeStruct((R * V,), jnp.float32),
        compiler_params=pltpu.CompilerParams(needs_layout_passes=False),
        scratch_types=[
            pltpu.VMEM((K,), jnp.int32),
            pltpu.VMEM((NB * K,), jnp.float32),
            pltpu.VMEM((NB * K,), jnp.float32),
            pltpu.VMEM((NB * V,), jnp.float32),
            pltpu.VMEM((NB * V,), jnp.float32),
            pltpu.SemaphoreType.DMA,
            pltpu.SemaphoreType.DMA,
            pltpu.SemaphoreType.DMA,
            pltpu.SemaphoreType.DMA,
        ],
    )
    def k(xp_hbm, idx_hbm, out_hbm, idxv, s0, s1, ob0, ob1, ss0, ss1, so0, so1):
        wid = lax.axis_index("s") * NC + lax.axis_index("c")
        row0 = wid * rows_per_w
        b = row0 // (R // B)

        pltpu.sync_copy(idx_hbm.at[pl.ds(b * K, K)], idxv)

        pltpu.async_copy(xp_hbm.at[pl.ds(row0 * K, NB * K)], s0, ss0)
        pltpu.async_copy(xp_hbm.at[pl.ds((row0 + NB) * K, NB * K)], s1, ss1)

        z = jnp.zeros((L,), jnp.float32)

        def zero_body(it, _):
            base = it * (32 * L)
            for u in range(32):
                ob0[pl.ds(base + u * L, L)] = z
                ob1[pl.ds(base + u * L, L)] = z
            return _

        lax.fori_loop(0, (NB * V) // (32 * L), zero_body, None)

        def do_group(g, it, sbuf, ssem, obuf, osem):
            rbase = row0 + g * NB
            pltpu.make_async_copy(
                xp_hbm.at[pl.ds(rbase * K, NB * K)], sbuf, ssem).wait()
            @pl.when(it >= 1)
            def _():
                pltpu.make_async_copy(
                    obuf, out_hbm.at[pl.ds(rbase * V, NB * V)], osem).wait()
            for i in range(kc):
                iv = idxv[pl.ds(i * L, L)]
                for j in range(NB):
                    sv = sbuf[pl.ds(j * K + i * L, L)]
                    plsc.store_scatter(obuf, [iv + j * V], sv)
            pltpu.async_copy(obuf, out_hbm.at[pl.ds(rbase * V, NB * V)], osem)
            @pl.when(it < n_iters - 1)
            def _():
                pltpu.async_copy(
                    xp_hbm.at[pl.ds((rbase + 2 * NB) * K, NB * K)], sbuf, ssem)

        def body(it, _):
            do_group(2 * it, it, s0, ss0, ob0, so0)
            do_group(2 * it + 1, it, s1, ss1, ob1, so1)
            return _

        lax.fori_loop(0, n_iters, body, None)

        pltpu.make_async_copy(
            ob0, out_hbm.at[pl.ds(row0 * V, NB * V)], so0).wait()
        pltpu.make_async_copy(
            ob1, out_hbm.at[pl.ds(row0 * V, NB * V)], so1).wait()

    return k


def kernel(X_pooled, X_original, A, idx):
    B, C, T, V = X_original.shape
    K = X_pooled.shape[3]
    R = B * C * T
    xp = X_pooled.reshape(R * K)
    idxf = idx.reshape(B * K).astype(jnp.int32)
    out = _build_sc_kernel(R, K, V, NB=4, B=B)(xp, idxf)
    return out.reshape(B, C, T, V)


# speedup vs baseline: 363.9396x; 2.4536x over previous
import functools

import jax
import jax.numpy as jnp
from jax import lax
from jax.experimental import pallas as pl
from jax.experimental.pallas import tpu as pltpu
from jax.experimental.pallas import tpu_sc as plsc

L = 16
NB = 8


def _build_sc_kernel(B, C, T, K, V):
    info = plsc.get_sparse_core_info()
    NC, NS = info.num_cores, info.num_subcores
    NW = NC * NS
    n_groups = (B * C * T // NB) // NW
    n_iters = n_groups // 2
    tg = T // NB
    w_per_b = NW // B
    c_per_w = C // w_per_b
    kc = K // L

    mesh = plsc.VectorSubcoreMesh(core_axis_name="c", subcore_axis_name="s")

    @functools.partial(
        pl.kernel,
        mesh=mesh,
        out_type=jax.ShapeDtypeStruct((B, C, T, V), jnp.float32),
        compiler_params=pltpu.CompilerParams(needs_layout_passes=False),
        scratch_types=[
            pltpu.VMEM((K,), jnp.int32),
            pltpu.VMEM((NB, K), jnp.float32),
            pltpu.VMEM((NB, K), jnp.float32),
            pltpu.VMEM((NB, V), jnp.float32),
            pltpu.VMEM((NB, V), jnp.float32),
            pltpu.SemaphoreType.DMA,
            pltpu.SemaphoreType.DMA,
            pltpu.SemaphoreType.DMA,
            pltpu.SemaphoreType.DMA,
        ],
    )
    def k(xp_hbm, idx_hbm, out_hbm, idxv, s0, s1, ob0, ob1, ss0, ss1, so0, so1):
        wid = lax.axis_index("s") * NC + lax.axis_index("c")
        b = wid // w_per_b
        c0 = (wid % w_per_b) * c_per_w

        pltpu.sync_copy(idx_hbm.at[b], idxv)

        def src_slice(g):
            c = c0 + g // tg
            t0 = (g % tg) * NB
            return xp_hbm.at[b, c, pl.ds(t0, NB), :]

        def out_slice(g):
            c = c0 + g // tg
            t0 = (g % tg) * NB
            return out_hbm.at[b, c, pl.ds(t0, NB), :]

        pltpu.async_copy(src_slice(0), s0, ss0)
        pltpu.async_copy(src_slice(1), s1, ss1)

        z = jnp.zeros((L,), jnp.float32)

        def zero_body(it, _):
            col0 = it * (4 * L)
            for j in range(NB):
                for u in range(4):
                    ob0[j, pl.ds(col0 + u * L, L)] = z
                    ob1[j, pl.ds(col0 + u * L, L)] = z
            return _

        lax.fori_loop(0, V // (4 * L), zero_body, None)

        rowv = [jnp.full((L,), j, jnp.int32) for j in range(NB)]

        def do_group(g, it, sbuf, ssem, obuf, osem):
            pltpu.make_async_copy(src_slice(g), sbuf, ssem).wait()
            @pl.when(it >= 1)
            def _():
                pltpu.make_async_copy(obuf, out_slice(g), osem).wait()
            for i in range(kc):
                iv = idxv[pl.ds(i * L, L)]
                for j in range(NB):
                    sv = sbuf[j, pl.ds(i * L, L)]
                    plsc.store_scatter(obuf, [rowv[j], iv], sv)
            pltpu.async_copy(obuf, out_slice(g), osem)
            @pl.when(it < n_iters - 1)
            def _():
                pltpu.async_copy(src_slice(g + 2), sbuf, ssem)

        def body(it, _):
            do_group(2 * it, it, s0, ss0, ob0, so0)
            do_group(2 * it + 1, it, s1, ss1, ob1, so1)
            return _

        lax.fori_loop(0, n_iters, body, None)

        pltpu.make_async_copy(ob0, out_slice(0), so0).wait()
        pltpu.make_async_copy(ob1, out_slice(1), so1).wait()

    return k


def kernel(X_pooled, X_original, A, idx):
    B, C, T, V = X_original.shape
    K = X_pooled.shape[3]
    idx32 = idx.astype(jnp.int32)
    return _build_sc_kernel(B, C, T, K, V)(X_pooled, idx32)


# speedup vs baseline: 471.9497x; 1.2968x over previous
import functools

import jax
import jax.numpy as jnp
from jax import lax
from jax.experimental import pallas as pl
from jax.experimental.pallas import tpu as pltpu
from jax.experimental.pallas import tpu_sc as plsc

L = 16
NB = 8


def _build_sc_kernel(B, C, T, K, V):
    info = plsc.get_sparse_core_info()
    NC, NS = info.num_cores, info.num_subcores
    NW = NC * NS
    n_groups = (B * C * T // NB) // NW
    n_iters = n_groups // 2
    tg = T // NB
    w_per_b = NW // B
    c_per_w = C // w_per_b
    kc = K // L

    mesh = plsc.VectorSubcoreMesh(core_axis_name="c", subcore_axis_name="s")

    @functools.partial(
        pl.kernel,
        mesh=mesh,
        out_type=jax.ShapeDtypeStruct((B, C, T, V), jnp.float32),
        compiler_params=pltpu.CompilerParams(needs_layout_passes=False),
        scratch_types=[
            pltpu.VMEM((K,), jnp.int32),
            pltpu.VMEM((NB, K), jnp.float32),
            pltpu.VMEM((NB, K), jnp.float32),
            pltpu.VMEM((NB, V), jnp.float32),
            pltpu.VMEM((NB, V), jnp.float32),
            pltpu.SemaphoreType.DMA,
            pltpu.SemaphoreType.DMA,
            pltpu.SemaphoreType.DMA,
            pltpu.SemaphoreType.DMA,
        ],
    )
    def k(xp_hbm, idx_hbm, out_hbm, idxv, s0, s1, ob0, ob1, ss0, ss1, so0, so1):
        wid = lax.axis_index("s") * NC + lax.axis_index("c")
        b = wid // w_per_b
        c0 = (wid % w_per_b) * c_per_w

        pltpu.sync_copy(idx_hbm.at[b], idxv)

        def src_slice(g):
            c = c0 + g // tg
            t0 = (g % tg) * NB
            return xp_hbm.at[b, c, pl.ds(t0, NB), :]

        def out_slice(g):
            c = c0 + g // tg
            t0 = (g % tg) * NB
            return out_hbm.at[b, c, pl.ds(t0, NB), :]

        pltpu.async_copy(src_slice(0), s0, ss0)
        pltpu.async_copy(src_slice(1), s1, ss1)

        z = jnp.zeros((L,), jnp.float32)

        @plsc.parallel_loop(0, V // L, unroll=4)
        def _(it):
            col0 = it * L
            for j in range(NB):
                ob0[j, pl.ds(col0, L)] = z
                ob1[j, pl.ds(col0, L)] = z

        rowv = [jnp.full((L,), j, jnp.int32) for j in range(NB)]

        def do_group(g, it, sbuf, ssem, obuf, osem):
            pltpu.make_async_copy(src_slice(g), sbuf, ssem).wait()
            @pl.when(it >= 1)
            def _():
                pltpu.make_async_copy(obuf, out_slice(g), osem).wait()
            @plsc.parallel_loop(0, kc, unroll=8)
            def _(i):
                col = i * L
                iv = idxv[pl.ds(col, L)]
                for j in range(NB):
                    sv = sbuf[j, pl.ds(col, L)]
                    plsc.store_scatter(obuf, [rowv[j], iv], sv)
            pltpu.async_copy(obuf, out_slice(g), osem)
            @pl.when(it < n_iters - 1)
            def _():
                pltpu.async_copy(src_slice(g + 2), sbuf, ssem)

        def body(it, _):
            do_group(2 * it, it, s0, ss0, ob0, so0)
            do_group(2 * it + 1, it, s1, ss1, ob1, so1)
            return _

        lax.fori_loop(0, n_iters, body, None)

        pltpu.make_async_copy(ob0, out_slice(0), so0).wait()
        pltpu.make_async_copy(ob1, out_slice(1), so1).wait()

    return k


def kernel(X_pooled, X_original, A, idx):
    B, C, T, V = X_original.shape
    K = X_pooled.shape[3]
    idx32 = idx.astype(jnp.int32)
    return _build_sc_kernel(B, C, T, K, V)(X_pooled, idx32)
